# Initial kernel scaffold; baseline (speedup 1.0000x reference)
import jax
import jax.numpy as jnp
from jax.experimental import pallas as pl


def kernel(hidden_states, token_ids, token_to_expert, gate_proj, up_proj, down_proj):
    raise NotImplementedError("write your pallas kernel here")


# speedup vs baseline: 2.3897x; 2.3897x over previous
import functools

import jax
import jax.numpy as jnp
from jax import lax
from jax.experimental import pallas as pl
from jax.experimental.pallas import tpu as pltpu
from jax.experimental.pallas import tpu_sc as plsc

NE = 16
H = 768
EI = 192
T = 2048
TILE = 128
PADDED = 4096
NT = PADDED // TILE

NC, NS, L = 2, 16, 16
NW = NC * NS
TPW = T // NW
CHUNKS = TPW // L

_mesh = plsc.VectorSubcoreMesh(
    core_axis_name="c", subcore_axis_name="s", num_cores=NC, num_subcores=NS)


def _wid():
    return lax.axis_index("s") * NC + lax.axis_index("c")


def _iota():
    return lax.iota(jnp.int32, L)


def _vsum(v):
    return plsc.cumsum(v)[L - 1]


@functools.partial(
    pl.kernel,
    out_type=(
        jax.ShapeDtypeStruct((T,), jnp.int32),
        jax.ShapeDtypeStruct((NW, NE), jnp.int32),
    ),
    mesh=_mesh,
    compiler_params=pltpu.CompilerParams(needs_layout_passes=False),
    scratch_types=[
        pltpu.VMEM((TPW,), jnp.int32),
        pltpu.VMEM((TPW,), jnp.int32),
        pltpu.VMEM((NE,), jnp.int32),
        pltpu.SemaphoreType.DMA,
    ],
)
def _route1(tok_hbm, t2e_hbm, eid_hbm, hist_hbm, tok_v, eid_v, hist_v, sem):
    w = _wid()
    base = w * TPW
    pltpu.sync_copy(tok_hbm.at[pl.ds(base, TPW)], tok_v)
    vocab = t2e_hbm.shape[0]
    for k in range(CHUNKS):
        t = tok_v[pl.ds(k * L, L)]
        tok_v[pl.ds(k * L, L)] = jnp.clip(t, 0, vocab - 1)
    pltpu.async_copy(t2e_hbm.at[tok_v], eid_v, sem).wait()
    pltpu.sync_copy(eid_v, eid_hbm.at[pl.ds(base, TPW)])
    chunks = [eid_v[pl.ds(k * L, L)] for k in range(CHUNKS)]
    iota = _iota()
    hvec = jnp.zeros((L,), jnp.int32)
    for e in range(NE):
        acc = jnp.zeros((L,), jnp.int32)
        for ch in chunks:
            acc = acc + jnp.where(ch == e, 1, 0)
        cnt = _vsum(acc)
        hvec = jnp.where(iota == e, cnt, hvec)
    hist_v[...] = hvec
    pltpu.sync_copy(hist_v, hist_hbm.at[w])


@functools.partial(
    pl.kernel,
    out_type=(
        jax.ShapeDtypeStruct((PADDED, H), jnp.float32),
        jax.ShapeDtypeStruct((T,), jnp.int32),
        jax.ShapeDtypeStruct((NT,), jnp.int32),
    ),
    mesh=_mesh,
    compiler_params=pltpu.CompilerParams(needs_layout_passes=False),
    scratch_types=[
        pltpu.VMEM((TPW,), jnp.int32),
        pltpu.VMEM((NW, NE), jnp.int32),
        pltpu.VMEM((TPW,), jnp.int32),
        pltpu.VMEM((NT,), jnp.int32),
        pltpu.VMEM((TPW, H), jnp.float32),
        pltpu.SemaphoreType.DMA,
    ],
)
def _route2(eid_hbm, hist_hbm, x_hbm, xpad_hbm, slot_hbm, te_hbm,
            eid_v, hist_v, slot_v, te_v, rows_v, sem):
    w = _wid()
    base = w * TPW
    pltpu.sync_copy(eid_hbm.at[pl.ds(base, TPW)], eid_v)
    pltpu.sync_copy(hist_hbm, hist_v)
    iota = _iota()

    counts_v = jnp.zeros((L,), jnp.int32)
    prior_v = jnp.zeros((L,), jnp.int32)
    for e in range(NE):
        esp = jnp.full((L,), e, jnp.int32)
        v0 = plsc.load_gather(hist_v, [iota, esp])
        v1 = plsc.load_gather(hist_v, [iota + L, esp])
        cnt = _vsum(v0) + _vsum(v1)
        pr = (_vsum(jnp.where(iota < w, v0, 0))
              + _vsum(jnp.where(iota + L < w, v1, 0)))
        counts_v = jnp.where(iota == e, cnt, counts_v)
        prior_v = jnp.where(iota == e, pr, prior_v)

    pad_v = (counts_v + (TILE - 1)) & ~jnp.int32(TILE - 1)
    incl = plsc.cumsum(pad_v)
    base_v = incl - pad_v
    comb_v = base_v + prior_v

    chunks = [eid_v[pl.ds(k * L, L)] for k in range(CHUNKS)]
    run_v = jnp.zeros((L,), jnp.int32)
    for k, ch in enumerate(chunks):
        combg = comb_v.at[ch].get(mode="promise_in_bounds")
        rung = run_v.at[ch].get(mode="promise_in_bounds")
        rank = jnp.zeros((L,), jnp.int32)
        for e in range(NE):
            mi = jnp.where(ch == e, 1, 0)
            cs = plsc.cumsum(mi)
            rank = rank + mi * (cs - mi)
            run_v = run_v + jnp.where(iota == e, cs[L - 1], 0)
        slot_v[pl.ds(k * L, L)] = combg + rung + rank
    pltpu.sync_copy(slot_v, slot_hbm.at[pl.ds(base, TPW)])

    pltpu.sync_copy(x_hbm.at[pl.ds(base, TPW)], rows_v)
    pltpu.async_copy(rows_v, xpad_hbm.at[slot_v], sem).wait()

    @pl.when(w == 0)
    def _():
        for j in range(NT // L):
            jvec = (iota + j * L) * TILE
            acc = jnp.zeros((L,), jnp.int32)
            for e in range(NE):
                acc = acc + jnp.where(incl[e] <= jvec, 1, 0)
            te_v[pl.ds(j * L, L)] = jnp.minimum(acc, NE - 1)
        pltpu.sync_copy(te_v, te_hbm)


def _mlp_body(te_ref, x_ref, wg_ref, wu_ref, wd_ref, o_ref):
    x = x_ref[...]
    g = jax.lax.dot_general(x, wg_ref[0], (((1,), (0,)), ((), ())),
                            preferred_element_type=jnp.float32)
    u = jax.lax.dot_general(x, wu_ref[0], (((1,), (0,)), ((), ())),
                            preferred_element_type=jnp.float32)
    h = g * jax.lax.logistic(g) * u
    o_ref[...] = jax.lax.dot_general(h, wd_ref[0], (((1,), (0,)), ((), ())),
                                     preferred_element_type=jnp.float32)


def _grouped_mlp(te, x_pad, gate, up, down):
    return pl.pallas_call(
        _mlp_body,
        grid_spec=pltpu.PrefetchScalarGridSpec(
            num_scalar_prefetch=1,
            grid=(NT,),
            in_specs=[
                pl.BlockSpec((TILE, H), lambda i, te_ref: (i, 0)),
                pl.BlockSpec((1, H, EI), lambda i, te_ref: (te_ref[i], 0, 0)),
                pl.BlockSpec((1, H, EI), lambda i, te_ref: (te_ref[i], 0, 0)),
                pl.BlockSpec((1, EI, H), lambda i, te_ref: (te_ref[i], 0, 0)),
            ],
            out_specs=pl.BlockSpec((TILE, H), lambda i, te_ref: (i, 0)),
        ),
        out_shape=jax.ShapeDtypeStruct((PADDED, H), jnp.float32),
    )(te, x_pad, gate, up, down)


@functools.partial(
    pl.kernel,
    out_type=jax.ShapeDtypeStruct((T, H), jnp.float32),
    mesh=_mesh,
    compiler_params=pltpu.CompilerParams(needs_layout_passes=False),
    scratch_types=[
        pltpu.VMEM((TPW,), jnp.int32),
        pltpu.VMEM((TPW, H), jnp.float32),
        pltpu.SemaphoreType.DMA,
    ],
)
def _unsort(slot_hbm, opad_hbm, out_hbm, idx_v, rows_v, sem):
    w = _wid()
    base = w * TPW
    pltpu.sync_copy(slot_hbm.at[pl.ds(base, TPW)], idx_v)
    pltpu.async_copy(opad_hbm.at[idx_v], rows_v, sem).wait()
    pltpu.sync_copy(rows_v, out_hbm.at[pl.ds(base, TPW)])


def kernel(hidden_states, token_ids, token_to_expert, gate_proj, up_proj,
           down_proj):
    Bq, Sq, Hq = hidden_states.shape
    x2d = hidden_states.reshape(T, H)
    tok = token_ids.reshape(T)
    eids, hist = _route1(tok, token_to_expert)
    x_pad, slot, te = _route2(eids, hist, x2d)
    o_pad = _grouped_mlp(te, x_pad, gate_proj, up_proj, down_proj)
    out = _unsort(slot, o_pad)
    return out.reshape(Bq, Sq, Hq)


# speedup vs baseline: 2.3952x; 1.0023x over previous
import functools

import jax
import jax.numpy as jnp
from jax import lax
from jax.experimental import pallas as pl
from jax.experimental.pallas import tpu as pltpu
from jax.experimental.pallas import tpu_sc as plsc

NE = 16
H = 768
EI = 192
T = 2048
TILE = 128
PADDED = 4096
NT = PADDED // TILE

NC, NS, L = 2, 16, 16
NW = NC * NS
TPW = T // NW
CHUNKS = TPW // L

_mesh = plsc.VectorSubcoreMesh(
    core_axis_name="c", subcore_axis_name="s", num_cores=NC, num_subcores=NS)


def _wid():
    return lax.axis_index("s") * NC + lax.axis_index("c")


def _iota():
    return lax.iota(jnp.int32, L)


def _vsum(v):
    return plsc.cumsum(v)[L - 1]


@functools.partial(
    pl.kernel,
    out_type=(
        jax.ShapeDtypeStruct((T,), jnp.int32),
        jax.ShapeDtypeStruct((NW, NE), jnp.int32),
    ),
    mesh=_mesh,
    compiler_params=pltpu.CompilerParams(needs_layout_passes=False),
    scratch_types=[
        pltpu.VMEM((TPW,), jnp.int32),
        pltpu.VMEM((TPW,), jnp.int32),
        pltpu.VMEM((NE,), jnp.int32),
        pltpu.SemaphoreType.DMA,
    ],
)
def _route1(tok_hbm, t2e_hbm, eid_hbm, hist_hbm, tok_v, eid_v, hist_v, sem):
    w = _wid()
    base = w * TPW
    pltpu.sync_copy(tok_hbm.at[pl.ds(base, TPW)], tok_v)
    vocab = t2e_hbm.shape[0]
    for k in range(CHUNKS):
        t = tok_v[pl.ds(k * L, L)]
        tok_v[pl.ds(k * L, L)] = jnp.clip(t, 0, vocab - 1)
    pltpu.async_copy(t2e_hbm.at[tok_v], eid_v, sem).wait()
    pltpu.sync_copy(eid_v, eid_hbm.at[pl.ds(base, TPW)])
    chunks = [eid_v[pl.ds(k * L, L)] for k in range(CHUNKS)]
    iota = _iota()
    hvec = jnp.zeros((L,), jnp.int32)
    for e in range(NE):
        acc = jnp.zeros((L,), jnp.int32)
        for ch in chunks:
            acc = acc + jnp.where(ch == e, 1, 0)
        cnt = _vsum(acc)
        hvec = jnp.where(iota == e, cnt, hvec)
    hist_v[...] = hvec
    pltpu.sync_copy(hist_v, hist_hbm.at[w])


@functools.partial(
    pl.kernel,
    out_type=(
        jax.ShapeDtypeStruct((PADDED, H), jnp.float32),
        jax.ShapeDtypeStruct((T,), jnp.int32),
        jax.ShapeDtypeStruct((NT,), jnp.int32),
    ),
    mesh=_mesh,
    compiler_params=pltpu.CompilerParams(needs_layout_passes=False),
    scratch_types=[
        pltpu.VMEM((TPW,), jnp.int32),
        pltpu.VMEM((NW, NE), jnp.int32),
        pltpu.VMEM((TPW,), jnp.int32),
        pltpu.VMEM((NT,), jnp.int32),
        pltpu.VMEM((TPW, H), jnp.float32),
        pltpu.SemaphoreType.DMA,
    ],
)
def _route2(eid_hbm, hist_hbm, x_hbm, xpad_hbm, slot_hbm, te_hbm,
            eid_v, hist_v, slot_v, te_v, rows_v, sem):
    w = _wid()
    base = w * TPW
    pltpu.sync_copy(eid_hbm.at[pl.ds(base, TPW)], eid_v)
    pltpu.sync_copy(hist_hbm, hist_v)
    iota = _iota()

    counts_v = jnp.zeros((L,), jnp.int32)
    prior_v = jnp.zeros((L,), jnp.int32)
    for e in range(NE):
        esp = jnp.full((L,), e, jnp.int32)
        v0 = plsc.load_gather(hist_v, [iota, esp])
        v1 = plsc.load_gather(hist_v, [iota + L, esp])
        cnt = _vsum(v0) + _vsum(v1)
        pr = (_vsum(jnp.where(iota < w, v0, 0))
              + _vsum(jnp.where(iota + L < w, v1, 0)))
        counts_v = jnp.where(iota == e, cnt, counts_v)
        prior_v = jnp.where(iota == e, pr, prior_v)

    pad_v = (counts_v + (TILE - 1)) & ~jnp.int32(TILE - 1)
    incl = plsc.cumsum(pad_v)
    base_v = incl - pad_v
    comb_v = base_v + prior_v

    chunks = [eid_v[pl.ds(k * L, L)] for k in range(CHUNKS)]
    run_v = jnp.zeros((L,), jnp.int32)
    for k, ch in enumerate(chunks):
        combg = comb_v.at[ch].get(mode="promise_in_bounds")
        rung = run_v.at[ch].get(mode="promise_in_bounds")
        rank = jnp.zeros((L,), jnp.int32)
        for e in range(NE):
            mi = jnp.where(ch == e, 1, 0)
            cs = plsc.cumsum(mi)
            rank = rank + mi * (cs - mi)
            run_v = run_v + jnp.where(iota == e, cs[L - 1], 0)
        slot_v[pl.ds(k * L, L)] = combg + rung + rank
    pltpu.sync_copy(slot_v, slot_hbm.at[pl.ds(base, TPW)])

    pltpu.sync_copy(x_hbm.at[pl.ds(base, TPW)], rows_v)
    pltpu.async_copy(rows_v, xpad_hbm.at[slot_v], sem).wait()

    @pl.when(w == 0)
    def _():
        for j in range(NT // L):
            jvec = (iota + j * L) * TILE
            acc = jnp.zeros((L,), jnp.int32)
            for e in range(NE):
                acc = acc + jnp.where(incl[e] <= jvec, 1, 0)
            te_v[pl.ds(j * L, L)] = jnp.minimum(acc, NE - 1)
        pltpu.sync_copy(te_v, te_hbm)


def _mlp_body(te_ref, x_ref, wg_ref, wu_ref, wd_ref, o_ref):
    x = x_ref[...].astype(jnp.bfloat16)
    g = jax.lax.dot_general(x, wg_ref[0].astype(jnp.bfloat16),
                            (((1,), (0,)), ((), ())),
                            preferred_element_type=jnp.float32)
    u = jax.lax.dot_general(x, wu_ref[0].astype(jnp.bfloat16),
                            (((1,), (0,)), ((), ())),
                            preferred_element_type=jnp.float32)
    h = g * jax.lax.logistic(g) * u
    o_ref[...] = jax.lax.dot_general(h.astype(jnp.bfloat16),
                                     wd_ref[0].astype(jnp.bfloat16),
                                     (((1,), (0,)), ((), ())),
                                     preferred_element_type=jnp.float32)


def _grouped_mlp(te, x_pad, gate, up, down):
    return pl.pallas_call(
        _mlp_body,
        grid_spec=pltpu.PrefetchScalarGridSpec(
            num_scalar_prefetch=1,
            grid=(NT,),
            in_specs=[
                pl.BlockSpec((TILE, H), lambda i, te_ref: (i, 0)),
                pl.BlockSpec((1, H, EI), lambda i, te_ref: (te_ref[i], 0, 0)),
                pl.BlockSpec((1, H, EI), lambda i, te_ref: (te_ref[i], 0, 0)),
                pl.BlockSpec((1, EI, H), lambda i, te_ref: (te_ref[i], 0, 0)),
            ],
            out_specs=pl.BlockSpec((TILE, H), lambda i, te_ref: (i, 0)),
        ),
        out_shape=jax.ShapeDtypeStruct((PADDED, H), jnp.float32),
    )(te, x_pad, gate, up, down)


@functools.partial(
    pl.kernel,
    out_type=jax.ShapeDtypeStruct((T, H), jnp.float32),
    mesh=_mesh,
    compiler_params=pltpu.CompilerParams(needs_layout_passes=False),
    scratch_types=[
        pltpu.VMEM((TPW,), jnp.int32),
        pltpu.VMEM((TPW, H), jnp.float32),
        pltpu.SemaphoreType.DMA,
    ],
)
def _unsort(slot_hbm, opad_hbm, out_hbm, idx_v, rows_v, sem):
    w = _wid()
    base = w * TPW
    pltpu.sync_copy(slot_hbm.at[pl.ds(base, TPW)], idx_v)
    pltpu.async_copy(opad_hbm.at[idx_v], rows_v, sem).wait()
    pltpu.sync_copy(rows_v, out_hbm.at[pl.ds(base, TPW)])


def kernel(hidden_states, token_ids, token_to_expert, gate_proj, up_proj,
           down_proj):
    Bq, Sq, Hq = hidden_states.shape
    x2d = hidden_states.reshape(T, H)
    tok = token_ids.reshape(T)
    eids, hist = _route1(tok, token_to_expert)
    x_pad, slot, te = _route2(eids, hist, x2d)
    o_pad = _grouped_mlp(te, x_pad, gate_proj, up_proj, down_proj)
    out = _unsort(slot, o_pad)
    return out.reshape(Bq, Sq, Hq)
